# baseline (speedup 1.0000x reference)
import functools

import jax
import jax.numpy as jnp
from jax import lax
from jax.experimental import pallas as pl
from jax.experimental.pallas import tpu as pltpu
from jax.experimental.pallas import tpu_sc as plsc

N_NODES = 10000
N_EDGES = 320000
D = 128

NC = 2
NS = 16
NT = NC * NS
CH = 128
NCH = -(-N_EDGES // (NT * CH))
EPT = NCH * CH
ACC_ROWS = N_NODES + 8
ROWS_PER_TILE = 624
REM_ROWS = N_NODES - NS * ROWS_PER_TILE
HALF = CH // 2


def _sc_body(x_hbm, dst_hbm, src_hbm, zeros_hbm, out_hbm,
             idx_s, db0, db1, rows0, rows1, acc,
             semz, semd0, semd1, semr0a, semr0b, semr1a, semr1b):
    c = lax.axis_index("c")
    s = lax.axis_index("s")
    tid = c * NS + s

    zdst = acc.at[pl.ds(s * ROWS_PER_TILE, ROWS_PER_TILE)]
    pltpu.async_copy(zeros_hbm.at[pl.ds(0, ROWS_PER_TILE)], zdst, semz)

    pltpu.async_copy(dst_hbm.at[tid, pl.ds(0, 1)], db0, semd0)
    pltpu.async_copy(dst_hbm.at[tid, pl.ds(1, 1)], db1, semd1)
    pltpu.sync_copy(src_hbm.at[tid], idx_s)

    def _start_gather(j, rows, sa, sb):
        pltpu.async_copy(x_hbm.at[idx_s.at[pl.ds(j * CH, HALF)]],
                         rows.at[pl.ds(0, HALF)], sa)
        pltpu.async_copy(x_hbm.at[idx_s.at[pl.ds(j * CH + HALF, HALF)]],
                         rows.at[pl.ds(HALF, HALF)], sb)

    def _wait_gather(rows, sa, sb):
        pltpu.make_async_copy(x_hbm.at[idx_s.at[pl.ds(0, HALF)]],
                              rows.at[pl.ds(0, HALF)], sa).wait()
        pltpu.make_async_copy(x_hbm.at[idx_s.at[pl.ds(0, HALF)]],
                              rows.at[pl.ds(HALF, HALF)], sb).wait()

    def _wait_dst(db, semd):
        pltpu.make_async_copy(dst_hbm.at[tid, pl.ds(0, 1)], db, semd).wait()

    _start_gather(0, rows0, semr0a, semr0b)

    pltpu.make_async_copy(zeros_hbm.at[pl.ds(0, ROWS_PER_TILE)], zdst,
                          semz).wait()

    @pl.when(s == 0)
    def _():
        pltpu.sync_copy(zeros_hbm.at[pl.ds(0, REM_ROWS)],
                        acc.at[pl.ds(NS * ROWS_PER_TILE, REM_ROWS)])

    plsc.subcore_barrier()

    @pl.loop(0, (NCH - 1) // 2)
    def _(i):
        j = 2 * i
        _wait_gather(rows0, semr0a, semr0b)
        _start_gather(j + 1, rows1, semr1a, semr1b)
        _wait_dst(db0, semd0)
        pltpu.sync_copy(rows0, acc.at[db0.at[0]], add=True)
        pltpu.async_copy(dst_hbm.at[tid, pl.ds(j + 2, 1)], db0, semd0)

        _wait_gather(rows1, semr1a, semr1b)
        _start_gather(j + 2, rows0, semr0a, semr0b)
        _wait_dst(db1, semd1)
        pltpu.sync_copy(rows1, acc.at[db1.at[0]], add=True)
        pltpu.async_copy(dst_hbm.at[tid, pl.ds(j + 3, 1)], db1, semd1)

    _wait_gather(rows0, semr0a, semr0b)
    _wait_dst(db0, semd0)
    pltpu.sync_copy(rows0, acc.at[db0.at[0]], add=True)
    _wait_dst(db1, semd1)

    plsc.subcore_barrier()
    sl = pl.ds(s * ROWS_PER_TILE, ROWS_PER_TILE)
    pltpu.sync_copy(acc.at[sl], out_hbm.at[c, sl])

    @pl.when(s == 0)
    def _():
        sl2 = pl.ds(NS * ROWS_PER_TILE, REM_ROWS)
        pltpu.sync_copy(acc.at[sl2], out_hbm.at[c, sl2])


_sc_scatter = functools.partial(
    pl.kernel,
    out_type=jax.ShapeDtypeStruct((NC, N_NODES, D), jnp.float32),
    mesh=plsc.VectorSubcoreMesh(core_axis_name="c", subcore_axis_name="s"),
    scratch_types=[
        pltpu.VMEM((EPT,), jnp.int32),
        pltpu.VMEM((1, CH), jnp.int32),
        pltpu.VMEM((1, CH), jnp.int32),
        pltpu.VMEM((CH, D), jnp.float32),
        pltpu.VMEM((CH, D), jnp.float32),
        pltpu.VMEM_SHARED((ACC_ROWS, D), jnp.float32),
    ] + [pltpu.SemaphoreType.DMA] * 7,
)(_sc_body)


def _add_body(p_ref, q_ref, o_ref):
    o_ref[...] = p_ref[0] + q_ref[0]


def _tc_add(partials):
    blk = 1000
    return pl.pallas_call(
        _add_body,
        grid=(N_NODES // blk,),
        in_specs=[
            pl.BlockSpec((1, blk, D), lambda i: (0, i, 0)),
            pl.BlockSpec((1, blk, D), lambda i: (1, i, 0)),
        ],
        out_specs=pl.BlockSpec((blk, D), lambda i: (i, 0)),
        out_shape=jax.ShapeDtypeStruct((N_NODES, D), jnp.float32),
    )(partials, partials)


@jax.jit
def kernel(x, edge_index):
    pad = NT * EPT - N_EDGES
    dst = jnp.pad(edge_index[0], (0, pad), constant_values=N_NODES)
    src = jnp.pad(edge_index[1], (0, pad), constant_values=0)
    dst = jnp.pad(dst.reshape(NT, NCH, CH), ((0, 0), (0, 2), (0, 0)))
    src = src.reshape(NT, EPT)
    zeros = jnp.zeros((ROWS_PER_TILE, D), jnp.float32)
    partials = _sc_scatter(x, dst, src, zeros)
    return _tc_add(partials)


# speedup vs baseline: 1.8124x; 1.8124x over previous
import functools

import jax
import jax.numpy as jnp
from jax import lax
from jax.experimental import pallas as pl
from jax.experimental.pallas import tpu as pltpu
from jax.experimental.pallas import tpu_sc as plsc

N_NODES = 10000
N_EDGES = 320000
D = 128

NC = 2
NS = 16
CH = 80
EDGES_PER_TILE = N_EDGES // (NC * NS)
NCH = EDGES_PER_TILE // CH
ROWS_PER_TILE = 624
REM_ROWS = N_NODES - NS * ROWS_PER_TILE


def _sc_body(x_hbm, dst_hbm, src_hbm, zeros_hbm, out_hbm,
             idx_d, idx_s, rows0, rows1, acc,
             semz, sem0a, sem0b, sem0c, sem0d, sem1a, sem1b, sem1c, sem1d):
    c = lax.axis_index("c")
    s = lax.axis_index("s")
    tid = c * NS + s

    zdst = acc.at[pl.ds(s * ROWS_PER_TILE, ROWS_PER_TILE)]
    pltpu.async_copy(zeros_hbm.at[pl.ds(0, ROWS_PER_TILE)], zdst, semz)

    pltpu.sync_copy(dst_hbm.at[tid], idx_d)
    pltpu.sync_copy(src_hbm.at[tid], idx_s)

    SPLITS = ((0, 24), (24, 24), (48, 16), (64, 16))

    def _start_gather(j, rows, sems):
        for q, (off, ln) in enumerate(SPLITS):
            pltpu.async_copy(x_hbm.at[idx_s.at[pl.ds(j * CH + off, ln)]],
                             rows.at[pl.ds(off, ln)], sems[q])

    def _wait_gather(rows, sems):
        for q, (off, ln) in enumerate(SPLITS):
            pltpu.make_async_copy(x_hbm.at[idx_s.at[pl.ds(0, ln)]],
                                  rows.at[pl.ds(off, ln)], sems[q]).wait()

    s0 = (sem0a, sem0b, sem0c, sem0d)
    s1 = (sem1a, sem1b, sem1c, sem1d)

    _start_gather(0, rows0, s0)

    pltpu.make_async_copy(zeros_hbm.at[pl.ds(0, ROWS_PER_TILE)], zdst,
                          semz).wait()

    @pl.when(s == 0)
    def _():
        pltpu.sync_copy(zeros_hbm.at[pl.ds(0, REM_ROWS)],
                        acc.at[pl.ds(NS * ROWS_PER_TILE, REM_ROWS)])

    plsc.subcore_barrier()

    @pl.loop(0, (NCH - 1) // 2)
    def _(i):
        j = 2 * i
        _wait_gather(rows0, s0)
        _start_gather(j + 1, rows1, s1)
        pltpu.sync_copy(rows0, acc.at[idx_d.at[j]], add=True)
        _wait_gather(rows1, s1)
        _start_gather(j + 2, rows0, s0)
        pltpu.sync_copy(rows1, acc.at[idx_d.at[j + 1]], add=True)

    _wait_gather(rows0, s0)
    pltpu.sync_copy(rows0, acc.at[idx_d.at[NCH - 1]], add=True)

    plsc.subcore_barrier()
    sl = pl.ds(s * ROWS_PER_TILE, ROWS_PER_TILE)
    pltpu.sync_copy(acc.at[sl], out_hbm.at[c, sl])

    @pl.when(s == 0)
    def _():
        sl2 = pl.ds(NS * ROWS_PER_TILE, REM_ROWS)
        pltpu.sync_copy(acc.at[sl2], out_hbm.at[c, sl2])


_sc_scatter = functools.partial(
    pl.kernel,
    out_type=jax.ShapeDtypeStruct((NC, N_NODES, D), jnp.float32),
    mesh=plsc.VectorSubcoreMesh(core_axis_name="c", subcore_axis_name="s"),
    scratch_types=[
        pltpu.VMEM((NCH, CH), jnp.int32),
        pltpu.VMEM((EDGES_PER_TILE,), jnp.int32),
        pltpu.VMEM((CH, D), jnp.float32),
        pltpu.VMEM((CH, D), jnp.float32),
        pltpu.VMEM_SHARED((N_NODES, D), jnp.float32),
    ] + [pltpu.SemaphoreType.DMA] * 9,
)(_sc_body)


def _add_body(p_ref, q_ref, o_ref):
    o_ref[...] = p_ref[0] + q_ref[0]


def _tc_add(partials):
    blk = 1000
    return pl.pallas_call(
        _add_body,
        grid=(N_NODES // blk,),
        in_specs=[
            pl.BlockSpec((1, blk, D), lambda i: (0, i, 0)),
            pl.BlockSpec((1, blk, D), lambda i: (1, i, 0)),
        ],
        out_specs=pl.BlockSpec((blk, D), lambda i: (i, 0)),
        out_shape=jax.ShapeDtypeStruct((N_NODES, D), jnp.float32),
    )(partials, partials)


@jax.jit
def kernel(x, edge_index):
    dst = edge_index[0].reshape(NC * NS, NCH, CH)
    src = edge_index[1].reshape(NC * NS, EDGES_PER_TILE)
    zeros = jnp.zeros((ROWS_PER_TILE, D), jnp.float32)
    partials = _sc_scatter(x, dst, src, zeros)
    return _tc_add(partials)


# speedup vs baseline: 1.8314x; 1.0105x over previous
import functools

import jax
import jax.numpy as jnp
from jax import lax
from jax.experimental import pallas as pl
from jax.experimental.pallas import tpu as pltpu
from jax.experimental.pallas import tpu_sc as plsc

N_NODES = 10000
N_EDGES = 320000
D = 128

NC = 2
NS = 16
CH = 80
EDGES_PER_TILE = N_EDGES // (NC * NS)
NCH = EDGES_PER_TILE // CH
ROWS_PER_TILE = 624
REM_ROWS = N_NODES - NS * ROWS_PER_TILE


def _sc_body(x_hbm, dst_hbm, src_hbm, zeros_hbm, out_hbm,
             idx_d, idx_s, rows0, rows1, acc,
             semz, sem0a, sem0b, sem0c, sem0d, sem1a, sem1b, sem1c, sem1d):
    c = lax.axis_index("c")
    s = lax.axis_index("s")
    tid = c * NS + s

    zdst = acc.at[pl.ds(s * ROWS_PER_TILE, ROWS_PER_TILE)]
    pltpu.async_copy(zeros_hbm.at[pl.ds(0, ROWS_PER_TILE)], zdst, semz)

    pltpu.sync_copy(dst_hbm.at[tid], idx_d)
    pltpu.sync_copy(src_hbm.at[tid], idx_s)

    SPLITS = ((0, 24), (24, 24), (48, 16), (64, 16))

    def _start_gather(j, rows, sems):
        for q, (off, ln) in enumerate(SPLITS):
            pltpu.async_copy(x_hbm.at[idx_s.at[pl.ds(j * CH + off, ln)]],
                             rows.at[pl.ds(off, ln)], sems[q])

    def _wait_gather(rows, sems):
        for q, (off, ln) in enumerate(SPLITS):
            pltpu.make_async_copy(x_hbm.at[idx_s.at[pl.ds(0, ln)]],
                                  rows.at[pl.ds(off, ln)], sems[q]).wait()

    s0 = (sem0a, sem0b, sem0c, sem0d)
    s1 = (sem1a, sem1b, sem1c, sem1d)

    _start_gather(0, rows0, s0)

    pltpu.make_async_copy(zeros_hbm.at[pl.ds(0, ROWS_PER_TILE)], zdst,
                          semz).wait()

    @pl.when(s == 0)
    def _():
        pltpu.sync_copy(zeros_hbm.at[pl.ds(0, REM_ROWS)],
                        acc.at[pl.ds(NS * ROWS_PER_TILE, REM_ROWS)])

    plsc.subcore_barrier()

    @pl.loop(0, (NCH - 1) // 2)
    def _(i):
        j = 2 * i
        _wait_gather(rows0, s0)
        _start_gather(j + 1, rows1, s1)
        pltpu.sync_copy(rows0, acc.at[idx_d.at[j]], add=True)
        _wait_gather(rows1, s1)
        _start_gather(j + 2, rows0, s0)
        pltpu.sync_copy(rows1, acc.at[idx_d.at[j + 1]], add=True)

    _wait_gather(rows0, s0)
    pltpu.sync_copy(rows0, acc.at[idx_d.at[NCH - 1]], add=True)

    plsc.subcore_barrier()
    sl = pl.ds(s * ROWS_PER_TILE, ROWS_PER_TILE)
    pltpu.sync_copy(acc.at[sl], out_hbm.at[c, sl])

    @pl.when(s == 0)
    def _():
        sl2 = pl.ds(NS * ROWS_PER_TILE, REM_ROWS)
        pltpu.sync_copy(acc.at[sl2], out_hbm.at[c, sl2])


_sc_scatter = functools.partial(
    pl.kernel,
    out_type=jax.ShapeDtypeStruct((NC, N_NODES, D), jnp.float32),
    mesh=plsc.VectorSubcoreMesh(core_axis_name="c", subcore_axis_name="s"),
    scratch_types=[
        pltpu.VMEM((NCH, CH), jnp.int32),
        pltpu.VMEM((EDGES_PER_TILE,), jnp.int32),
        pltpu.VMEM((CH, D), jnp.float32),
        pltpu.VMEM((CH, D), jnp.float32),
        pltpu.VMEM_SHARED((N_NODES, D), jnp.float32),
    ] + [pltpu.SemaphoreType.DMA] * 9,
)(_sc_body)


def _add_body(p_ref, q_ref, o_ref):
    o_ref[...] = p_ref[0] + q_ref[0]


def _tc_add(partials):
    blk = 2000
    return pl.pallas_call(
        _add_body,
        grid=(N_NODES // blk,),
        in_specs=[
            pl.BlockSpec((1, blk, D), lambda i: (0, i, 0)),
            pl.BlockSpec((1, blk, D), lambda i: (1, i, 0)),
        ],
        out_specs=pl.BlockSpec((blk, D), lambda i: (i, 0)),
        out_shape=jax.ShapeDtypeStruct((N_NODES, D), jnp.float32),
    )(partials, partials)


@jax.jit
def kernel(x, edge_index):
    dst = edge_index[0].reshape(NC * NS, NCH, CH)
    src = edge_index[1].reshape(NC * NS, EDGES_PER_TILE)
    zeros = jnp.zeros((ROWS_PER_TILE, D), jnp.float32)
    partials = _sc_scatter(x, dst, src, zeros)
    return _tc_add(partials)
